# baseline (speedup 1.0000x reference)
import functools

import jax
import jax.numpy as jnp
from jax import lax
from jax.experimental import pallas as pl
from jax.experimental.pallas import tpu as pltpu
from jax.experimental.pallas import tpu_sc as plsc

_VOCAB = 100000
_EMBED = 64
_N_FEAT = 36
_HIDDEN = 1024
_N_CLASSES = 3
_BATCH = 4096

_NC = 2
_NS = 16
_NW = _NC * _NS
_NP = _N_FEAT // 2
_BSTRIPE = _BATCH // _NW
_FPAD = 40


def _sc_gather_body(t_hbm, table_hbm, out_hbm, t_v, idx_v, buf, sem):
    wid = lax.axis_index("s") * _NC + lax.axis_index("c")
    b0 = wid * _BSTRIPE
    pltpu.sync_copy(t_hbm.at[pl.ds(b0, _BSTRIPE)], t_v)

    lane = lax.iota(jnp.int32, 16)

    def tbody(f, carry):
        col = jnp.full((16,), f, jnp.int32)
        for s in range(_BSTRIPE // 16):
            vals = plsc.load_gather(t_v, [lane + 16 * s, col])
            idx_v[f, pl.ds(16 * s, 16)] = vals
        return carry

    lax.fori_loop(0, _N_FEAT, tbody, 0)

    (al, ar), (bl, br) = buf
    sem_a, sem_b = sem

    def fire(p, dl, dr, s):
        pltpu.async_copy(table_hbm.at[idx_v.at[2 * p]], dl, s)
        pltpu.async_copy(table_hbm.at[idx_v.at[2 * p + 1]], dr, s)

    def drain_write(p, dl, dr, s):
        pltpu.make_async_copy(table_hbm.at[pl.ds(0, _BSTRIPE)], dl, s).wait()
        pltpu.make_async_copy(table_hbm.at[pl.ds(0, _BSTRIPE)], dr, s).wait()
        dst = out_hbm.at[pl.ds(p * _BATCH + b0, _BSTRIPE)]
        pltpu.sync_copy(dl, dst.at[:, 0:_EMBED])
        pltpu.sync_copy(dr, dst.at[:, _EMBED:128])

    fire(0, al, ar, sem_a)

    def body(k, carry):
        p = 2 * k
        fire(p + 1, bl, br, sem_b)
        drain_write(p, al, ar, sem_a)

        @pl.when(p + 2 < _NP)
        def _():
            fire(p + 2, al, ar, sem_a)

        drain_write(p + 1, bl, br, sem_b)
        return carry

    lax.fori_loop(0, _NP // 2, body, 0)


_sc_gather = functools.partial(
    pl.kernel,
    mesh=plsc.VectorSubcoreMesh(core_axis_name="c", subcore_axis_name="s"),
    out_type=jax.ShapeDtypeStruct((_NP * _BATCH, 128), jnp.float32),
    scratch_types=[
        pltpu.VMEM((_BSTRIPE, _N_FEAT), jnp.int32),
        pltpu.VMEM((_N_FEAT, _BSTRIPE), jnp.int32),
        ((pltpu.VMEM((_BSTRIPE, _EMBED), jnp.float32),
          pltpu.VMEM((_BSTRIPE, _EMBED), jnp.float32)),
         (pltpu.VMEM((_BSTRIPE, _EMBED), jnp.float32),
          pltpu.VMEM((_BSTRIPE, _EMBED), jnp.float32))),
        (pltpu.SemaphoreType.DMA, pltpu.SemaphoreType.DMA),
    ],
    compiler_params=pltpu.CompilerParams(use_tc_tiling_on_sc=False,
                                         needs_layout_passes=False),
)(_sc_gather_body)


_VCHUNK = 512
_NCH = (_VOCAB // _VCHUNK)
_VTAIL = _NCH * _VCHUNK


def _sc_detranspose_body(embt_hbm, tail_hbm, out_hbm, in_v, out_v):
    wid = lax.axis_index("s") * _NC + lax.axis_index("c")
    lane = lax.iota(jnp.int32, 16)
    parity64 = (lane % 2) * _EMBED
    rowbase = lane // 2

    def chunk(c):
        v0 = pl.multiple_of(c * _VCHUNK, _VCHUNK)
        pltpu.sync_copy(embt_hbm.at[:, pl.ds(v0, _VCHUNK)], in_v)

        def jbody(j, carry):
            rows = rowbase + 8 * j
            off = pl.multiple_of(16 * j, 16)
            for e in range(_EMBED):
                vals = in_v[e, pl.ds(off, 16)]
                plsc.store_scatter(out_v, [rows, parity64 + e], vals)
            return carry

        lax.fori_loop(0, _VCHUNK // 16, jbody, 0)
        pltpu.sync_copy(out_v, out_hbm.at[pl.ds(
            pl.multiple_of(v0 // 2, _VCHUNK // 2), _VCHUNK // 2)])

    def kbody(k, carry):
        c = wid + _NW * k

        @pl.when(c < _NCH)
        def _():
            chunk(c)

        return carry

    lax.fori_loop(0, (_NCH + _NW - 1) // _NW, kbody, 0)

    @pl.when(wid == 0)
    def _():
        pltpu.sync_copy(tail_hbm, out_hbm.at[pl.ds(_VTAIL // 2,
                                                   (_VOCAB - _VTAIL) // 2)])


_sc_detranspose = functools.partial(
    pl.kernel,
    mesh=plsc.VectorSubcoreMesh(core_axis_name="c", subcore_axis_name="s"),
    out_type=jax.ShapeDtypeStruct((_VOCAB // 2, 128), jnp.float32),
    scratch_types=[
        pltpu.VMEM((_EMBED, _VCHUNK), jnp.float32),
        pltpu.VMEM((_VCHUNK // 2, 128), jnp.float32),
    ],
    compiler_params=pltpu.CompilerParams(use_tc_tiling_on_sc=True,
                                         needs_layout_passes=False),
)(_sc_detranspose_body)


_BB = 512
_NT_DIMS = (((1,), (1,)), ((), ()))


def _tc_mlp_body(x_ref, w1_ref, b1_ref, w2_ref, b2_ref, out_ref):
    xb = jnp.transpose(x_ref[...].astype(jnp.bfloat16), (1, 0, 2)).reshape(
        _BB, _N_FEAT * _EMBED)
    acc = b1_ref[...] + jnp.dot(xb, w1_ref[...],
                                preferred_element_type=jnp.float32)
    h = jnp.maximum(acc, 0.0)
    out = lax.dot_general(h, w2_ref[...], _NT_DIMS,
                          preferred_element_type=jnp.float32)
    out_ref[...] = out + b2_ref[...]


def _tc_mlp(x4, w1, b1, w2_pad, b2_pad):
    return pl.pallas_call(
        _tc_mlp_body,
        grid=(_BATCH // _BB,),
        in_specs=[
            pl.BlockSpec((_NP, _BB, 128), lambda i: (0, i, 0)),
            pl.BlockSpec((_N_FEAT * _EMBED, _HIDDEN), lambda i: (0, 0)),

            pl.BlockSpec((1, _HIDDEN), lambda i: (0, 0)),
            pl.BlockSpec((128, _HIDDEN), lambda i: (0, 0)),
            pl.BlockSpec((1, 128), lambda i: (0, 0)),
        ],
        out_specs=pl.BlockSpec((_BB, 128), lambda i: (i, 0)),
        out_shape=jax.ShapeDtypeStruct((_BATCH, 128), jnp.float32),
    )(x4, w1, b1, w2_pad, b2_pad)


def kernel(t, embeddings, W1, b1, W2, b2):
    tail = embeddings[_VTAIL:].reshape((_VOCAB - _VTAIL) // 2, 128)
    packed = _sc_detranspose(embeddings.T, tail)
    table_rm = packed.reshape(_VOCAB, _EMBED)
    rows = _sc_gather(t.astype(jnp.int32), table_rm)
    x4 = rows.reshape(_NP, _BATCH, 128)

    w2_pad = jnp.zeros((128, _HIDDEN), jnp.float32).at[:_N_CLASSES, :].set(W2)
    b2_pad = jnp.zeros((128,), jnp.float32).at[:_N_CLASSES].set(b2)
    logits = _tc_mlp(x4, W1.T.astype(jnp.bfloat16), b1.reshape(1, _HIDDEN),
                     w2_pad, b2_pad.reshape(1, 128))
    return logits[:, :_N_CLASSES]


# speedup vs baseline: 2.1280x; 2.1280x over previous
import functools

import jax
import jax.numpy as jnp
from jax import lax
from jax.experimental import pallas as pl
from jax.experimental.pallas import tpu as pltpu
from jax.experimental.pallas import tpu_sc as plsc

_VOCAB = 100000
_EMBED = 64
_N_FEAT = 36
_HIDDEN = 1024
_N_CLASSES = 3
_BATCH = 4096

_NC = 2
_NS = 16
_NW = _NC * _NS
_NP = _N_FEAT // 2
_BSTRIPE = _BATCH // _NW
_FPAD = 40


def _sc_gather_body(t_hbm, table_hbm, out_hbm, t_v, idx_v, buf, sem):
    wid = lax.axis_index("s") * _NC + lax.axis_index("c")
    b0 = wid * _BSTRIPE
    pltpu.sync_copy(t_hbm.at[pl.ds(b0, _BSTRIPE)], t_v)

    lane = lax.iota(jnp.int32, 16)

    def tbody(f, carry):
        col = jnp.full((16,), f, jnp.int32)
        for s in range(_BSTRIPE // 16):
            vals = plsc.load_gather(t_v, [lane + 16 * s, col])
            vals = jnp.where(
                vals < _SPLIT, 2 * vals,
                jnp.where(vals < 2 * _SPLIT, 2 * vals - (2 * _SPLIT - 1),
                          vals))
            idx_v[f, pl.ds(16 * s, 16)] = vals
        return carry

    lax.fori_loop(0, _N_FEAT, tbody, 0)

    (al, ar), (bl, br) = buf
    sem_a, sem_b = sem

    def fire(p, dl, dr, s):
        pltpu.async_copy(table_hbm.at[idx_v.at[2 * p]], dl, s)
        pltpu.async_copy(table_hbm.at[idx_v.at[2 * p + 1]], dr, s)

    def drain_write(p, dl, dr, s):
        pltpu.make_async_copy(table_hbm.at[pl.ds(0, _BSTRIPE)], dl, s).wait()
        pltpu.make_async_copy(table_hbm.at[pl.ds(0, _BSTRIPE)], dr, s).wait()
        dst = out_hbm.at[pl.ds(p * _BATCH + b0, _BSTRIPE)]
        pltpu.sync_copy(dl, dst.at[:, 0:_EMBED])
        pltpu.sync_copy(dr, dst.at[:, _EMBED:128])

    fire(0, al, ar, sem_a)

    def body(k, carry):
        p = 2 * k
        fire(p + 1, bl, br, sem_b)
        drain_write(p, al, ar, sem_a)

        @pl.when(p + 2 < _NP)
        def _():
            fire(p + 2, al, ar, sem_a)

        drain_write(p + 1, bl, br, sem_b)
        return carry

    lax.fori_loop(0, _NP // 2, body, 0)


_sc_gather = functools.partial(
    pl.kernel,
    mesh=plsc.VectorSubcoreMesh(core_axis_name="c", subcore_axis_name="s"),
    out_type=jax.ShapeDtypeStruct((_NP * _BATCH, 128), jnp.float32),
    scratch_types=[
        pltpu.VMEM((_BSTRIPE, _N_FEAT), jnp.int32),
        pltpu.VMEM((_N_FEAT, _BSTRIPE), jnp.int32),
        ((pltpu.VMEM((_BSTRIPE, _EMBED), jnp.float32),
          pltpu.VMEM((_BSTRIPE, _EMBED), jnp.float32)),
         (pltpu.VMEM((_BSTRIPE, _EMBED), jnp.float32),
          pltpu.VMEM((_BSTRIPE, _EMBED), jnp.float32))),
        (pltpu.SemaphoreType.DMA, pltpu.SemaphoreType.DMA),
    ],
    compiler_params=pltpu.CompilerParams(use_tc_tiling_on_sc=False,
                                         needs_layout_passes=False),
)(_sc_gather_body)


_PACK_BR = 4096
_NPBLK = 12
_SPLIT = _NPBLK * _PACK_BR
_NTAIL = _VOCAB - 2 * _SPLIT


def _tc_pack_body(ina_ref, inb_ref, tail_ref, out_ref):
    i = pl.program_id(0)

    @pl.when(i < _NPBLK)
    def _():
        out_ref[:, 0:_EMBED] = ina_ref[...].T
        out_ref[:, _EMBED:128] = inb_ref[...].T

    @pl.when(i == _NPBLK)
    def _():
        out_ref[0:_NTAIL // 2, :] = tail_ref[...]


def _tc_pack(embt, tail):
    return pl.pallas_call(
        _tc_pack_body,
        grid=(_NPBLK + 1,),
        in_specs=[
            pl.BlockSpec((_EMBED, _PACK_BR), lambda i: (0, i)),
            pl.BlockSpec((_EMBED, _PACK_BR), lambda i: (0, i + _NPBLK)),
            pl.BlockSpec((_NTAIL // 2, 128), lambda i: (0, 0)),
        ],
        out_specs=pl.BlockSpec((_PACK_BR, 128), lambda i: (i, 0)),
        out_shape=jax.ShapeDtypeStruct((_VOCAB // 2, 128), jnp.float32),
    )(embt, embt, tail)


_BB = 512
_NT_DIMS = (((1,), (1,)), ((), ()))


def _tc_mlp_body(x_ref, w1_ref, b1_ref, w2_ref, b2_ref, out_ref):
    xb = jnp.transpose(x_ref[...].astype(jnp.bfloat16), (1, 0, 2)).reshape(
        _BB, _N_FEAT * _EMBED)
    acc = b1_ref[...] + jnp.dot(xb, w1_ref[...],
                                preferred_element_type=jnp.float32)
    h = jnp.maximum(acc, 0.0)
    out = lax.dot_general(h, w2_ref[...], _NT_DIMS,
                          preferred_element_type=jnp.float32)
    out_ref[...] = out + b2_ref[...]


def _tc_mlp(x4, w1, b1, w2_pad, b2_pad):
    return pl.pallas_call(
        _tc_mlp_body,
        grid=(_BATCH // _BB,),
        in_specs=[
            pl.BlockSpec((_NP, _BB, 128), lambda i: (0, i, 0)),
            pl.BlockSpec((_N_FEAT * _EMBED, _HIDDEN), lambda i: (0, 0)),

            pl.BlockSpec((1, _HIDDEN), lambda i: (0, 0)),
            pl.BlockSpec((128, _HIDDEN), lambda i: (0, 0)),
            pl.BlockSpec((1, 128), lambda i: (0, 0)),
        ],
        out_specs=pl.BlockSpec((_BB, 128), lambda i: (i, 0)),
        out_shape=jax.ShapeDtypeStruct((_BATCH, 128), jnp.float32),
    )(x4, w1, b1, w2_pad, b2_pad)


def kernel(t, embeddings, W1, b1, W2, b2):
    tail = embeddings[2 * _SPLIT:].reshape(_NTAIL // 2, 128)
    packed = _tc_pack(embeddings.T, tail)
    table_rm = packed.reshape(_VOCAB, _EMBED)
    rows = _sc_gather(t.astype(jnp.int32), table_rm)
    x4 = rows.reshape(_NP, _BATCH, 128)

    w2_pad = jnp.zeros((128, _HIDDEN), jnp.float32).at[:_N_CLASSES, :].set(W2)
    b2_pad = jnp.zeros((128,), jnp.float32).at[:_N_CLASSES].set(b2)
    logits = _tc_mlp(x4, W1.T.astype(jnp.bfloat16), b1.reshape(1, _HIDDEN),
                     w2_pad, b2_pad.reshape(1, 128))
    return logits[:, :_N_CLASSES]
